# baseline (speedup 1.0000x reference)
import functools

import jax
import jax.numpy as jnp
from jax import lax
from jax.experimental import pallas as pl
from jax.experimental.pallas import tpu as pltpu
from jax.experimental.pallas import tpu_sc as plsc

NUM_CLASSES = 100000
EMBED_DIM = 64
MULT = 26
BATCH = 4096

B = BATCH * MULT
NC, NS = 2, 16
NW = NC * NS
BPW = B // NW
HALF = BPW // 2
GCH = 128
NG = HALF // GCH


def _emb_body(idx_hbm, table_hbm, out_hbm, idx_v, rows_v, sem):
    wid = lax.axis_index("s") * NC + lax.axis_index("c")
    base = wid * BPW
    pltpu.sync_copy(idx_hbm.at[pl.ds(base, BPW)], idx_v)

    lanes = lax.iota(jnp.int32, 16)

    def add_off(j, carry):
        p = lanes + j * 16
        off = lax.rem(p, MULT) * NUM_CLASSES
        idx_v[pl.ds(j * 16, 16)] = idx_v[pl.ds(j * 16, 16)] + off
        return carry

    lax.fori_loop(0, BPW // 16, add_off, 0)

    for h in range(2):
        hbase = h * HALF
        descs = [
            pltpu.async_copy(
                table_hbm.at[idx_v.at[pl.ds(hbase + c * GCH, GCH)]],
                rows_v.at[pl.ds(c * GCH, GCH)],
                sem,
            )
            for c in range(NG)
        ]
        for d in descs:
            d.wait()
        pltpu.sync_copy(rows_v, out_hbm.at[pl.ds(base + hbase, HALF)])


def kernel(to_emb, table):
    idx_flat = to_emb.reshape(B).astype(jnp.int32)
    mesh = plsc.VectorSubcoreMesh(core_axis_name="c", subcore_axis_name="s")
    run = functools.partial(
        pl.kernel,
        mesh=mesh,
        out_type=jax.ShapeDtypeStruct((B, EMBED_DIM), jnp.float32),
        scratch_types=[
            pltpu.VMEM((BPW,), jnp.int32),
            pltpu.VMEM((HALF, EMBED_DIM), jnp.float32),
            pltpu.SemaphoreType.DMA,
        ],
        compiler_params=pltpu.CompilerParams(use_tc_tiling_on_sc=False),
    )(_emb_body)
    out = run(idx_flat, table)
    return out.reshape(BATCH, MULT, EMBED_DIM)


# speedup vs baseline: 1.1045x; 1.1045x over previous
import functools

import jax
import jax.numpy as jnp
from jax import lax
from jax.experimental import pallas as pl
from jax.experimental.pallas import tpu as pltpu
from jax.experimental.pallas import tpu_sc as plsc

NUM_CLASSES = 100000
EMBED_DIM = 64
MULT = 26
BATCH = 4096

B = BATCH * MULT
NC, NS = 2, 16
NW = NC * NS
BPW = B // NW
PADD = 128
GCH = 128
NG = 2
SLOT = NG * GCH
NSLOT = BPW // SLOT


def _emb_body(idx_hbm, table_hbm, out_hbm, idx_v, rows0, rows1, g0, g1):
    wid = lax.axis_index("s") * NC + lax.axis_index("c")
    base = wid * BPW
    pltpu.sync_copy(idx_hbm.at[pl.ds(base, BPW)], idx_v)

    lanes = lax.iota(jnp.int32, 16)

    def add_off(j, carry):
        p = lanes + j * 16
        off = lax.rem(p, MULT) * NUM_CLASSES
        idx_v[pl.ds(j * 16, 16)] = idx_v[pl.ds(j * 16, 16)] + off
        return carry

    lax.fori_loop(0, BPW // 16, add_off, 0)

    rows = (rows0, rows1)
    gsem = (g0, g1)

    def writeback(s):
        pltpu.sync_copy(rows[s % 2], out_hbm.at[pl.ds(base + s * SLOT, SLOT)])

    gd = [None] * NSLOT
    for s in range(NSLOT):
        buf = rows[s % 2]
        gd[s] = [
            pltpu.async_copy(
                table_hbm.at[idx_v.at[pl.ds(s * SLOT + c * GCH, GCH)]],
                buf.at[pl.ds(c * GCH, GCH)],
                gsem[s % 2],
            )
            for c in range(NG)
        ]
        if s >= 1:
            for d in gd[s - 1]:
                d.wait()
            writeback(s - 1)
    for d in gd[NSLOT - 1]:
        d.wait()
    writeback(NSLOT - 1)


def kernel(to_emb, table):
    idx_flat = to_emb.reshape(B).astype(jnp.int32)
    tpad = jnp.pad(table, ((0, 0), (0, PADD - EMBED_DIM)))
    mesh = plsc.VectorSubcoreMesh(core_axis_name="c", subcore_axis_name="s")
    run = functools.partial(
        pl.kernel,
        mesh=mesh,
        out_type=jax.ShapeDtypeStruct((B, PADD), jnp.float32),
        scratch_types=[
            pltpu.VMEM((BPW,), jnp.int32),
            pltpu.VMEM((SLOT, PADD), jnp.float32),
            pltpu.VMEM((SLOT, PADD), jnp.float32),
            pltpu.SemaphoreType.DMA,
            pltpu.SemaphoreType.DMA,
        ],
    )(_emb_body)
    out = run(idx_flat, tpad)
    return out[:, :EMBED_DIM].reshape(BATCH, MULT, EMBED_DIM)
